# Initial kernel scaffold; baseline (speedup 1.0000x reference)
import jax
import jax.numpy as jnp
from jax.experimental import pallas as pl


def kernel(probabilities, hidden, edge_index, W, b):
    raise NotImplementedError("write your pallas kernel here")


# speedup vs baseline: 6.5876x; 6.5876x over previous
import functools

import jax
import jax.numpy as jnp
from jax import lax
from jax.experimental import pallas as pl
from jax.experimental.pallas import tpu as pltpu
from jax.experimental.pallas import tpu_sc as plsc

N_NODES = 10000
N_EDGES = 320000
HIDDEN = 128

NC = 2
NS = 16
NW = NC * NS

EPW = N_EDGES // NW
CHUNK = 200
NCHUNK = EPW // CHUNK
WB = 200
NBLK = N_NODES // WB

_mesh = plsc.VectorSubcoreMesh(
    core_axis_name="c", subcore_axis_name="s", num_cores=NC, num_subcores=NS
)


@functools.partial(
    pl.kernel,
    out_type=jax.ShapeDtypeStruct((NC, N_NODES, HIDDEN), jnp.float32),
    mesh=_mesh,
    scratch_types=[
        pltpu.VMEM((CHUNK,), jnp.int32),
        pltpu.VMEM((CHUNK,), jnp.int32),
        pltpu.VMEM((CHUNK,), jnp.float32),
        pltpu.VMEM((CHUNK, HIDDEN), jnp.float32),
        pltpu.VMEM_SHARED((N_NODES, HIDDEN), jnp.float32),
        pltpu.SemaphoreType.DMA,
    ],
)
def _sc_weighted_scatter(src_hbm, dst_hbm, prob_hbm, hidden_hbm, out_hbm,
                         dsti_v, srci_v, prob_v, rows_v, agg_sh, sem):
    c = lax.axis_index("c")
    s = lax.axis_index("s")
    wid = s * NC + c

    zvec = jnp.zeros((16,), jnp.float32)

    def _zrow(i, carry):
        for j in range(HIDDEN // 16):
            rows_v[i, pl.ds(j * 16, 16)] = zvec
        return carry

    lax.fori_loop(0, WB, _zrow, 0)
    for rep in range((NBLK + NS - 1) // NS):
        blk = rep * NS + s

        @pl.when(blk < NBLK)
        def _zero_blk():
            pltpu.sync_copy(rows_v.at[pl.ds(0, WB)],
                            agg_sh.at[pl.ds(blk * WB, WB)])

    plsc.subcore_barrier()

    ebase = wid * EPW

    def _chunk(ci, carry):
        off = ebase + ci * CHUNK
        pltpu.sync_copy(dst_hbm.at[pl.ds(off, CHUNK)], dsti_v)
        pltpu.sync_copy(src_hbm.at[pl.ds(off, CHUNK)], srci_v)
        pltpu.sync_copy(prob_hbm.at[pl.ds(off, CHUNK)], prob_v)
        pltpu.async_copy(hidden_hbm.at[dsti_v], rows_v, sem).wait()

        def _grp(g, gcarry):
            pvec = prob_v[pl.ds(g * 16, 16)]
            for l in range(16):
                p = pvec[l]
                e = g * 16 + l
                for j in range(HIDDEN // 16):
                    sl = pl.ds(j * 16, 16)
                    rows_v[e, sl] = rows_v[e, sl] * p
            return gcarry

        lax.fori_loop(0, CHUNK // 16, _grp, 0)
        rem = CHUNK - (CHUNK // 16) * 16
        if rem:
            pvec = prob_v[pl.ds(CHUNK - 16, 16)]
            for l in range(16 - rem, 16):
                p = pvec[l]
                e = CHUNK - 16 + l
                for j in range(HIDDEN // 16):
                    sl = pl.ds(j * 16, 16)
                    rows_v[e, sl] = rows_v[e, sl] * p
        pltpu.sync_copy(rows_v, agg_sh.at[srci_v], add=True)
        return carry

    lax.fori_loop(0, NCHUNK, _chunk, 0)
    plsc.subcore_barrier()

    for rep in range((NBLK + NS - 1) // NS):
        blk = rep * NS + s

        @pl.when(blk < NBLK)
        def _write_blk():
            pltpu.sync_copy(agg_sh.at[pl.ds(blk * WB, WB)],
                            out_hbm.at[c, pl.ds(blk * WB, WB)])


_BS = 1000


def _tc_linear_body(a_ref, w_ref, b_ref, o_ref):
    x = a_ref[0] + a_ref[1]
    y = lax.dot_general(x, w_ref[...], (((1,), (1,)), ((), ())),
                        preferred_element_type=jnp.float32)
    o_ref[...] = y + b_ref[...]


_tc_linear = pl.pallas_call(
    _tc_linear_body,
    grid=(N_NODES // _BS,),
    in_specs=[
        pl.BlockSpec((NC, _BS, HIDDEN), lambda i: (0, i, 0)),
        pl.BlockSpec((HIDDEN, HIDDEN), lambda i: (0, 0)),
        pl.BlockSpec((1, HIDDEN), lambda i: (0, 0)),
    ],
    out_specs=pl.BlockSpec((_BS, HIDDEN), lambda i: (i, 0)),
    out_shape=jax.ShapeDtypeStruct((N_NODES, HIDDEN), jnp.float32),
)


def kernel(probabilities, hidden, edge_index, W, b):
    ei = edge_index.astype(jnp.int32)
    src = ei[0]
    dst = ei[1]
    agg2 = _sc_weighted_scatter(src, dst, probabilities, hidden)
    return _tc_linear(agg2, W, b.reshape(1, HIDDEN))
